# Initial kernel scaffold; baseline (speedup 1.0000x reference)
import jax
import jax.numpy as jnp
from jax.experimental import pallas as pl


def kernel(nodes, neigh_idx, features_table, weight, weight_2, z):
    raise NotImplementedError("write your pallas kernel here")


# speedup vs baseline: 1.1868x; 1.1868x over previous
import functools

import jax
import jax.numpy as jnp
from jax import lax
from jax.experimental import pallas as pl
from jax.experimental.pallas import tpu as pltpu
from jax.experimental.pallas import tpu_sc as plsc

B = 8192
D = 256
S = 10
H = 1024
E = 256

NW = 32
NODES_PER_W = B // NW
NCHUNK = 8
NIDX = NCHUNK * S
CHUNKS = NODES_PER_W // NCHUNK
NODE_CH = 128


def _sc_body(nodes_hbm, nidx_hbm, table_hbm, nfeat_hbm, nmean_hbm,
             idx_v, rows_v, acc_v, nodeidx_v, noderows_v, sem, sem2):
    cid = lax.axis_index("c")
    sid = lax.axis_index("s")
    wid = sid * 2 + cid
    base = wid * NODES_PER_W

    for j in range(NODES_PER_W // NODE_CH):
        off = pl.multiple_of(base + j * NODE_CH, 8)
        pltpu.sync_copy(nodes_hbm.at[pl.ds(off, NODE_CH)], nodeidx_v)
        pltpu.async_copy(table_hbm.at[nodeidx_v], noderows_v, sem2).wait()
        pltpu.sync_copy(noderows_v, nfeat_hbm.at[pl.ds(off, NODE_CH)])

    def chunk_body(c, carry):
        row0 = base + c * NCHUNK
        ioff = pl.multiple_of(row0 * S, 8)
        pltpu.sync_copy(nidx_hbm.at[pl.ds(ioff, NIDX)], idx_v)
        pltpu.async_copy(table_hbm.at[idx_v], rows_v, sem).wait()
        for n in range(NCHUNK):
            for d in range(D // 16):
                sl = pl.ds(d * 16, 16)
                a = rows_v[n * S, sl]
                for s in range(1, S):
                    a = a + rows_v[n * S + s, sl]
                acc_v[n, sl] = a * jnp.float32(1.0 / S)
        pltpu.sync_copy(acc_v, nmean_hbm.at[pl.ds(pl.multiple_of(row0, 8), NCHUNK)])
        return carry

    lax.fori_loop(0, CHUNKS, chunk_body, 0)


@jax.jit
def _sc_gather(nodes, nidx_flat, table):
    mesh = plsc.VectorSubcoreMesh(core_axis_name="c", subcore_axis_name="s")
    f = pl.kernel(
        _sc_body,
        out_type=(
            jax.ShapeDtypeStruct((B, D), jnp.float32),
            jax.ShapeDtypeStruct((B, D), jnp.float32),
        ),
        mesh=mesh,
        scratch_types=[
            pltpu.VMEM((NIDX,), jnp.int32),
            pltpu.VMEM((NIDX, D), jnp.float32),
            pltpu.VMEM((NCHUNK, D), jnp.float32),
            pltpu.VMEM((NODE_CH,), jnp.int32),
            pltpu.VMEM((NODE_CH, D), jnp.float32),
            pltpu.SemaphoreType.DMA,
            pltpu.SemaphoreType.DMA,
        ],
    )
    return f(nodes, nidx_flat, table)


BLK = 1024
NBLK = B // BLK


def _phase1_body(xn_ref, xm_ref, w2_ref, zt_ref, out_ref, acc_ref):
    i = pl.program_id(0)

    @pl.when(i == 0)
    def _init():
        acc_ref[0] = jnp.float32(0.0)
        acc_ref[1] = jnp.float32(0.0)

    zt = zt_ref[...]
    tn = jnp.tanh(jnp.dot(xn_ref[...], w2_ref[...],
                          preferred_element_type=jnp.float32))
    tm = jnp.tanh(jnp.dot(xm_ref[...], w2_ref[...],
                          preferred_element_type=jnp.float32))
    acc_ref[0] += jnp.sum(tn * zt)
    acc_ref[1] += jnp.sum(tm * zt)

    @pl.when(i == NBLK - 1)
    def _fin():
        out_ref[0] = acc_ref[0] / B
        out_ref[1] = acc_ref[1] / B


@jax.jit
def _phase1(nfeat, nmean, w2, zt):
    return pl.pallas_call(
        _phase1_body,
        grid=(NBLK,),
        in_specs=[
            pl.BlockSpec((BLK, D), lambda i: (i, 0)),
            pl.BlockSpec((BLK, D), lambda i: (i, 0)),
            pl.BlockSpec((D, H), lambda i: (0, 0)),
            pl.BlockSpec((1, H), lambda i: (0, 0)),
        ],
        out_specs=pl.BlockSpec(memory_space=pltpu.SMEM),
        out_shape=jax.ShapeDtypeStruct((2,), jnp.float32),
        scratch_shapes=[pltpu.SMEM((2,), jnp.float32)],
    )(nfeat, nmean, w2, zt)


def _phase2_body(s_ref, xn_ref, xm_ref, w_ref, out_ref):
    u0 = s_ref[0]
    u1 = s_ref[1]
    m = jnp.maximum(u0, u1)
    e0 = jnp.exp(u0 - m)
    e1 = jnp.exp(u1 - m)
    a0 = e0 / (e0 + e1)
    a1 = e1 / (e0 + e1)
    comb = jnp.maximum(a0 * xn_ref[...] + a1 * xm_ref[...], 0.0)
    out_ref[...] = jnp.maximum(
        lax.dot_general(w_ref[...], comb, (((1,), (1,)), ((), ())),
                        preferred_element_type=jnp.float32),
        0.0)


@jax.jit
def _phase2(scal, nfeat, nmean, w):
    return pl.pallas_call(
        _phase2_body,
        grid=(NBLK,),
        in_specs=[
            pl.BlockSpec(memory_space=pltpu.SMEM),
            pl.BlockSpec((BLK, D), lambda i: (i, 0)),
            pl.BlockSpec((BLK, D), lambda i: (i, 0)),
            pl.BlockSpec((E, D), lambda i: (0, 0)),
        ],
        out_specs=pl.BlockSpec((E, BLK), lambda i: (0, i)),
        out_shape=jax.ShapeDtypeStruct((E, B), jnp.float32),
    )(scal, nfeat, nmean, w)


def kernel(nodes, neigh_idx, features_table, weight, weight_2, z):
    nodes = nodes.astype(jnp.int32)
    nidx_flat = neigh_idx.astype(jnp.int32).reshape(-1)
    nfeat, nmean = _sc_gather(nodes, nidx_flat, features_table)
    scal = _phase1(nfeat, nmean, weight_2, z.reshape(1, H))
    out = _phase2(scal, nfeat, nmean, weight)
    return out


# speedup vs baseline: 2.1427x; 1.8054x over previous
import functools

import jax
import jax.numpy as jnp
from jax import lax
from jax.experimental import pallas as pl
from jax.experimental.pallas import tpu as pltpu
from jax.experimental.pallas import tpu_sc as plsc

B = 8192
D = 256
S = 10
H = 1024
E = 256

NW = 32
NS = 16
NODES_PER_W = B // NW
CH = 128
NCHUNKS = NODES_PER_W * S // CH
NODE_CH = NODES_PER_W // CH


def _sc_body(nodes_hbm, nidx_hbm, table_hbm, nfeat_hbm, nrows_hbm,
             nidx_v, nodeidx_v, buf_v, sem, sem2):
    cid = lax.axis_index("c")
    sid = lax.axis_index("s")
    wid = sid * 2 + cid
    base = wid * NODES_PER_W

    pltpu.sync_copy(nodes_hbm.at[wid], nodeidx_v)
    pltpu.sync_copy(nidx_hbm.at[wid], nidx_v)

    def gather(j, bb):
        if j < NODE_CH:
            return pltpu.async_copy(table_hbm.at[nodeidx_v.at[j]],
                                    buf_v.at[bb], sem if bb == 0 else sem2)
        return pltpu.async_copy(table_hbm.at[nidx_v.at[j - NODE_CH]],
                                buf_v.at[bb], sem if bb == 0 else sem2)

    def drain(j, bb):
        if j < NODE_CH:
            pltpu.sync_copy(buf_v.at[bb],
                            nfeat_hbm.at[pl.ds(base + j * CH, CH)])
        else:
            s, h = divmod(j - NODE_CH, NODE_CH)
            pltpu.sync_copy(buf_v.at[bb],
                            nrows_hbm.at[s].at[pl.ds(base + h * CH, CH)])

    total = NODE_CH + NCHUNKS
    cp = gather(0, 0)
    for j in range(total):
        cp.wait()
        if j + 1 < total:
            cp = gather(j + 1, (j + 1) % 2)
        drain(j, j % 2)


@jax.jit
def _sc_gather(nodes2d, nidx2d, table):
    mesh = plsc.VectorSubcoreMesh(core_axis_name="c", subcore_axis_name="s")
    f = pl.kernel(
        _sc_body,
        out_type=(
            jax.ShapeDtypeStruct((B, D), jnp.float32),
            jax.ShapeDtypeStruct((S, B, D), jnp.float32),
        ),
        mesh=mesh,
        scratch_types=[
            pltpu.VMEM((NCHUNKS, CH), jnp.int32),
            pltpu.VMEM((NODE_CH, CH), jnp.int32),
            pltpu.VMEM((2, CH, D), jnp.float32),
            pltpu.SemaphoreType.DMA,
            pltpu.SemaphoreType.DMA,
        ],
    )
    return f(nodes2d, nidx2d, table)


BLK = 1024
NBLK = B // BLK


def _phase1_body(xn_ref, xr_ref, w2_ref, zt_ref, out_ref, mean_ref, acc_ref):
    i = pl.program_id(0)

    @pl.when(i == 0)
    def _init():
        acc_ref[0] = jnp.float32(0.0)
        acc_ref[1] = jnp.float32(0.0)

    zt = zt_ref[...]
    xm = xr_ref[0]
    for s in range(1, S):
        xm = xm + xr_ref[s]
    xm = xm * jnp.float32(1.0 / S)
    mean_ref[...] = xm
    tn = jnp.tanh(jnp.dot(xn_ref[...], w2_ref[...],
                          preferred_element_type=jnp.float32))
    tm = jnp.tanh(jnp.dot(xm, w2_ref[...],
                          preferred_element_type=jnp.float32))
    acc_ref[0] += jnp.sum(tn * zt)
    acc_ref[1] += jnp.sum(tm * zt)

    @pl.when(i == NBLK - 1)
    def _fin():
        out_ref[0] = acc_ref[0] / B
        out_ref[1] = acc_ref[1] / B


@jax.jit
def _phase1(nfeat, nrows, w2, zt):
    return pl.pallas_call(
        _phase1_body,
        grid=(NBLK,),
        in_specs=[
            pl.BlockSpec((BLK, D), lambda i: (i, 0)),
            pl.BlockSpec((S, BLK, D), lambda i: (0, i, 0)),
            pl.BlockSpec((D, H), lambda i: (0, 0)),
            pl.BlockSpec((1, H), lambda i: (0, 0)),
        ],
        out_specs=[
            pl.BlockSpec(memory_space=pltpu.SMEM),
            pl.BlockSpec((BLK, D), lambda i: (i, 0)),
        ],
        out_shape=[
            jax.ShapeDtypeStruct((2,), jnp.float32),
            jax.ShapeDtypeStruct((B, D), jnp.float32),
        ],
        scratch_shapes=[pltpu.SMEM((2,), jnp.float32)],
    )(nfeat, nrows, w2, zt)


def _phase2_body(s_ref, xn_ref, xm_ref, w_ref, out_ref):
    u0 = s_ref[0]
    u1 = s_ref[1]
    m = jnp.maximum(u0, u1)
    e0 = jnp.exp(u0 - m)
    e1 = jnp.exp(u1 - m)
    a0 = e0 / (e0 + e1)
    a1 = e1 / (e0 + e1)
    comb = jnp.maximum(a0 * xn_ref[...] + a1 * xm_ref[...], 0.0)
    out_ref[...] = jnp.maximum(
        lax.dot_general(w_ref[...], comb, (((1,), (1,)), ((), ())),
                        preferred_element_type=jnp.float32),
        0.0)


@jax.jit
def _phase2(scal, nfeat, nmean, w):
    return pl.pallas_call(
        _phase2_body,
        grid=(NBLK,),
        in_specs=[
            pl.BlockSpec(memory_space=pltpu.SMEM),
            pl.BlockSpec((BLK, D), lambda i: (i, 0)),
            pl.BlockSpec((BLK, D), lambda i: (i, 0)),
            pl.BlockSpec((E, D), lambda i: (0, 0)),
        ],
        out_specs=pl.BlockSpec((E, BLK), lambda i: (0, i)),
        out_shape=jax.ShapeDtypeStruct((E, B), jnp.float32),
    )(scal, nfeat, nmean, w)


def kernel(nodes, neigh_idx, features_table, weight, weight_2, z):
    nodes2d = nodes.astype(jnp.int32).reshape(NW, NODE_CH, CH)
    nidx2d = (neigh_idx.astype(jnp.int32)
              .reshape(NW, NODE_CH, CH, S)
              .transpose(0, 3, 1, 2)
              .reshape(NW, NCHUNKS, CH))
    nfeat, nrows = _sc_gather(nodes2d, nidx2d, features_table)
    scal, nmean = _phase1(nfeat, nrows, weight_2, z.reshape(1, H))
    out = _phase2(scal, nfeat, nmean, weight)
    return out
